# baseline (speedup 1.0000x reference)
import functools

import jax
import jax.numpy as jnp
from jax import lax
from jax.experimental import pallas as pl
from jax.experimental.pallas import tpu as pltpu
from jax.experimental.pallas import tpu_sc as plsc

E = 16
T = 2048
D = 1024
H = 1024
K = 2
N = T * K
BT = 256
NB = N // BT
MAXT = NB + E - 1

_NC = 2
_NS = 16
_NW = _NC * _NS


def _routing_metadata(weights, indices):
    eid = indices.reshape(-1).astype(jnp.int32)
    order = jnp.argsort(eid, stable=True).astype(jnp.int32)
    tok = (order // K).astype(jnp.int32)
    sw = weights.reshape(-1)[order]
    inv = jnp.argsort(order).astype(jnp.int32)
    sorted_eid = eid[order]
    offs = jnp.searchsorted(
        sorted_eid, jnp.arange(E + 1, dtype=jnp.int32), side="left",
        method="compare_all",
    ).astype(jnp.int32)
    counts = offs[1:] - offs[:E]
    fb = offs[:E] // BT
    lb = jnp.maximum(offs[1:] - 1, 0) // BT
    tiles_per = jnp.where(counts > 0, lb - fb + 1, 0).astype(jnp.int32)
    toffs = jnp.concatenate(
        [jnp.zeros((1,), jnp.int32), jnp.cumsum(tiles_per).astype(jnp.int32)])
    total = toffs[E]
    ti = jnp.arange(MAXT, dtype=jnp.int32)
    e_raw = jnp.clip(
        jnp.searchsorted(toffs, ti, side="right",
                         method="compare_all").astype(jnp.int32) - 1,
        0, E - 1)
    b_raw = fb[e_raw] + (ti - toffs[e_raw])
    valid = ti < total
    e_last = jnp.max(jnp.where(counts > 0, jnp.arange(E, dtype=jnp.int32), -1))
    eid_t = jnp.where(valid, e_raw, e_last).astype(jnp.int32)
    blk_t = jnp.where(valid, b_raw, NB - 1).astype(jnp.int32)
    lo_g = jnp.maximum(offs[eid_t], blk_t * BT)
    hi_g = jnp.minimum(offs[eid_t + 1], (blk_t + 1) * BT)
    lo_t = jnp.where(valid, lo_g - blk_t * BT, 0).astype(jnp.int32)
    hi_t = jnp.where(valid, hi_g - blk_t * BT, 0).astype(jnp.int32)
    first_t = jnp.concatenate(
        [jnp.ones((1,), jnp.int32),
         (blk_t[1:] != blk_t[:-1]).astype(jnp.int32)])
    return tok, sw, inv, blk_t, eid_t, lo_t, hi_t, first_t


def _sc_gather(table, idx):
    B = idx.shape[0]
    Dd = table.shape[1]
    bpw = B // _NW
    ch = 32
    nch = bpw // ch
    mesh = plsc.VectorSubcoreMesh(core_axis_name="c", subcore_axis_name="s")

    @functools.partial(
        pl.kernel,
        mesh=mesh,
        out_type=jax.ShapeDtypeStruct((B, Dd), table.dtype),
        scratch_types=[
            pltpu.VMEM((bpw,), jnp.int32),
            pltpu.VMEM((ch, Dd), table.dtype),
            pltpu.VMEM((ch, Dd), table.dtype),
            pltpu.SemaphoreType.DMA,
            pltpu.SemaphoreType.DMA,
            pltpu.SemaphoreType.DMA,
            pltpu.SemaphoreType.DMA,
        ],
    )
    def k(table_hbm, idx_hbm, out_hbm, idx_v, buf0, buf1, sg0, sg1, sw0, sw1):
        wid = lax.axis_index("s") * _NC + lax.axis_index("c")
        base = wid * bpw
        bufs = (buf0, buf1)
        sgs = (sg0, sg1)
        sws = (sw0, sw1)
        pltpu.sync_copy(idx_hbm.at[pl.ds(base, bpw)], idx_v)
        writes = [None, None]
        for c in range(nch):
            s = c % 2
            if writes[s] is not None:
                writes[s].wait()
            pltpu.async_copy(
                table_hbm.at[idx_v.at[pl.ds(c * ch, ch)]], bufs[s], sgs[s]
            ).wait()
            writes[s] = pltpu.async_copy(
                bufs[s], out_hbm.at[pl.ds(base + c * ch, ch)], sws[s])
        writes[0].wait()
        writes[1].wait()

    return k(table, idx)


def _sc_pair_add(ysw, inv):
    tpw = T // _NW
    ct = 16
    nch = tpw // ct
    nvec = ct * (D // 16)
    mesh = plsc.VectorSubcoreMesh(core_axis_name="c", subcore_axis_name="s")

    @functools.partial(
        pl.kernel,
        mesh=mesh,
        out_type=jax.ShapeDtypeStruct((T, D), jnp.float32),
        scratch_types=[
            pltpu.VMEM((2 * tpw,), jnp.int32),
            pltpu.VMEM((2 * ct, D), jnp.float32),
            pltpu.VMEM((2 * ct, D), jnp.float32),
            pltpu.VMEM((ct, D), jnp.float32),
            pltpu.VMEM((ct, D), jnp.float32),
            pltpu.SemaphoreType.DMA,
            pltpu.SemaphoreType.DMA,
            pltpu.SemaphoreType.DMA,
            pltpu.SemaphoreType.DMA,
        ],
    )
    def k(ysw_hbm, inv_hbm, y_hbm, idx_v, buf0, buf1, ob0, ob1,
          sg0, sg1, sw0, sw1):
        wid = lax.axis_index("s") * _NC + lax.axis_index("c")
        tbase = wid * tpw
        bufs = (buf0, buf1)
        obs = (ob0, ob1)
        sgs = (sg0, sg1)
        sws = (sw0, sw1)
        pltpu.sync_copy(inv_hbm.at[pl.ds(2 * tbase, 2 * tpw)], idx_v)
        gathers = [None, None]
        writes = [None, None]
        gathers[0] = pltpu.async_copy(
            ysw_hbm.at[idx_v.at[pl.ds(0, 2 * ct)]], buf0, sg0)
        for c in range(nch):
            s = c % 2
            gathers[s].wait()
            if c + 1 < nch:
                s1 = (c + 1) % 2
                gathers[s1] = pltpu.async_copy(
                    ysw_hbm.at[idx_v.at[pl.ds((c + 1) * 2 * ct, 2 * ct)]],
                    bufs[s1], sgs[s1])
            if writes[s] is not None:
                writes[s].wait()
            buf = bufs[s]
            ob = obs[s]

            def body(i, carry, buf=buf, ob=ob):
                j = i // 4
                vb = (i % 4) * 256
                for u in range(16):
                    v = vb + u * 16
                    ob[j, pl.ds(v, 16)] = (
                        buf[2 * j, pl.ds(v, 16)]
                        + buf[2 * j + 1, pl.ds(v, 16)])
                return carry

            lax.fori_loop(0, nvec // 16, body, 0)
            writes[s] = pltpu.async_copy(
                ob, y_hbm.at[pl.ds(tbase + c * ct, ct)], sws[s])
        writes[0].wait()
        writes[1].wait()

    return k(ysw, inv)


HCN = 2
HC = H // HCN


def _gmm_body(blk_r, eid_r, lo_r, hi_r, first_r,
              xs_r, w1y_r, w1g_r, w2_r, swb_r, out_r):
    i = pl.program_id(0)
    c = pl.program_id(1)
    lo = lo_r[i]
    hi = hi_r[i]
    rows = lax.broadcasted_iota(jnp.int32, (BT, 1), 0)
    mask = (rows >= lo) & (rows < hi)
    xb = jnp.where(mask, xs_r[...], 0.0)
    hy = jnp.dot(xb, w1y_r[0], preferred_element_type=jnp.float32)
    hg = jnp.dot(xb, w1g_r[0], preferred_element_type=jnp.float32)
    act = hy * (hg * jax.nn.sigmoid(hg))
    o = jnp.dot(act, w2_r[0], preferred_element_type=jnp.float32)
    o = o * swb_r[:, :1]
    init = (first_r[i] != 0) & (c == 0)

    @pl.when(init)
    def _():
        out_r[...] = o

    @pl.when(jnp.logical_not(init))
    def _():
        out_r[...] = out_r[...] + o


def _grouped_mlp(xs, W1, W2, swb, blk_t, eid_t, lo_t, hi_t, first_t):
    grid_spec = pltpu.PrefetchScalarGridSpec(
        num_scalar_prefetch=5,
        grid=(MAXT, HCN),
        in_specs=[
            pl.BlockSpec((BT, D), lambda i, c, b, e, l, h, f: (b[i], 0)),
            pl.BlockSpec((1, D, HC),
                         lambda i, c, b, e, l, h, f: (e[i], 0, c)),
            pl.BlockSpec((1, D, HC),
                         lambda i, c, b, e, l, h, f: (e[i], 0, HCN + c)),
            pl.BlockSpec((1, HC, D),
                         lambda i, c, b, e, l, h, f: (e[i], c, 0)),
            pl.BlockSpec((BT, 128), lambda i, c, b, e, l, h, f: (b[i], 0)),
        ],
        out_specs=pl.BlockSpec((BT, D), lambda i, c, b, e, l, h, f: (b[i], 0)),
    )
    return pl.pallas_call(
        _gmm_body,
        grid_spec=grid_spec,
        out_shape=jax.ShapeDtypeStruct((N, D), jnp.float32),
    )(blk_t, eid_t, lo_t, hi_t, first_t, xs, W1, W1, W2, swb)


def kernel(x, weights, indices, W1, W2):
    tok, sw, inv, blk_t, eid_t, lo_t, hi_t, first_t = _routing_metadata(
        weights, indices)
    swb = jnp.broadcast_to(sw[:, None], (N, 128))
    xs = _sc_gather(x, tok)
    ysw = _grouped_mlp(xs, W1, W2, swb, blk_t, eid_t, lo_t, hi_t, first_t)
    return _sc_pair_add(ysw, inv)


# speedup vs baseline: 1.1833x; 1.1833x over previous
import functools

import jax
import jax.numpy as jnp
from jax import lax
from jax.experimental import pallas as pl
from jax.experimental.pallas import tpu as pltpu
from jax.experimental.pallas import tpu_sc as plsc

E = 16
T = 2048
D = 1024
H = 1024
K = 2
N = T * K
BT = 256
NB = N // BT
MAXT = NB + E - 1

_NC = 2
_NS = 16
_NW = _NC * _NS


def _routing_metadata(weights, indices):
    eid = indices.reshape(-1).astype(jnp.int32)
    order = jnp.argsort(eid, stable=True).astype(jnp.int32)
    tok = (order // K).astype(jnp.int32)
    sw = weights.reshape(-1)[order]
    inv = jnp.argsort(order).astype(jnp.int32)
    offs = jnp.sum(
        eid[None, :] < jnp.arange(E + 1, dtype=jnp.int32)[:, None],
        axis=1, dtype=jnp.int32)
    counts = offs[1:] - offs[:E]
    fb = offs[:E] // BT
    lb = jnp.maximum(offs[1:] - 1, 0) // BT
    tiles_per = jnp.where(counts > 0, lb - fb + 1, 0).astype(jnp.int32)
    toffs = jnp.concatenate(
        [jnp.zeros((1,), jnp.int32), jnp.cumsum(tiles_per).astype(jnp.int32)])
    total = toffs[E]
    ti = jnp.arange(MAXT, dtype=jnp.int32)
    e_raw = jnp.clip(
        jnp.searchsorted(toffs, ti, side="right",
                         method="compare_all").astype(jnp.int32) - 1,
        0, E - 1)
    b_raw = fb[e_raw] + (ti - toffs[e_raw])
    valid = ti < total
    e_last = jnp.max(jnp.where(counts > 0, jnp.arange(E, dtype=jnp.int32), -1))
    eid_t = jnp.where(valid, e_raw, e_last).astype(jnp.int32)
    blk_t = jnp.where(valid, b_raw, NB - 1).astype(jnp.int32)
    lo_g = jnp.maximum(offs[eid_t], blk_t * BT)
    hi_g = jnp.minimum(offs[eid_t + 1], (blk_t + 1) * BT)
    lo_t = jnp.where(valid, lo_g - blk_t * BT, 0).astype(jnp.int32)
    hi_t = jnp.where(valid, hi_g - blk_t * BT, 0).astype(jnp.int32)
    first_t = jnp.concatenate(
        [jnp.ones((1,), jnp.int32),
         (blk_t[1:] != blk_t[:-1]).astype(jnp.int32)])
    return tok, sw, inv, blk_t, eid_t, lo_t, hi_t, first_t


def _sc_gather(table, idx):
    B = idx.shape[0]
    Dd = table.shape[1]
    bpw = B // _NW
    ch = 32
    nch = bpw // ch
    mesh = plsc.VectorSubcoreMesh(core_axis_name="c", subcore_axis_name="s")

    @functools.partial(
        pl.kernel,
        mesh=mesh,
        out_type=jax.ShapeDtypeStruct((B, Dd), table.dtype),
        scratch_types=[
            pltpu.VMEM((bpw,), jnp.int32),
            pltpu.VMEM((ch, Dd), table.dtype),
            pltpu.VMEM((ch, Dd), table.dtype),
            pltpu.SemaphoreType.DMA,
            pltpu.SemaphoreType.DMA,
            pltpu.SemaphoreType.DMA,
            pltpu.SemaphoreType.DMA,
        ],
    )
    def k(table_hbm, idx_hbm, out_hbm, idx_v, buf0, buf1, sg0, sg1, sw0, sw1):
        wid = lax.axis_index("s") * _NC + lax.axis_index("c")
        base = wid * bpw
        bufs = (buf0, buf1)
        sgs = (sg0, sg1)
        sws = (sw0, sw1)
        pltpu.sync_copy(idx_hbm.at[pl.ds(base, bpw)], idx_v)
        writes = [None, None]
        for c in range(nch):
            s = c % 2
            if writes[s] is not None:
                writes[s].wait()
            pltpu.async_copy(
                table_hbm.at[idx_v.at[pl.ds(c * ch, ch)]], bufs[s], sgs[s]
            ).wait()
            writes[s] = pltpu.async_copy(
                bufs[s], out_hbm.at[pl.ds(base + c * ch, ch)], sws[s])
        writes[0].wait()
        writes[1].wait()

    return k(table, idx)


def _sc_pair_add(ysw, inv):
    tpw = T // _NW
    ct = 16
    nch = tpw // ct
    nvec = ct * (D // 16)
    mesh = plsc.VectorSubcoreMesh(core_axis_name="c", subcore_axis_name="s")

    @functools.partial(
        pl.kernel,
        mesh=mesh,
        out_type=jax.ShapeDtypeStruct((T, D), jnp.float32),
        scratch_types=[
            pltpu.VMEM((2 * tpw,), jnp.int32),
            pltpu.VMEM((2 * ct, D), jnp.float32),
            pltpu.VMEM((2 * ct, D), jnp.float32),
            pltpu.VMEM((ct, D), jnp.float32),
            pltpu.VMEM((ct, D), jnp.float32),
            pltpu.SemaphoreType.DMA,
            pltpu.SemaphoreType.DMA,
            pltpu.SemaphoreType.DMA,
            pltpu.SemaphoreType.DMA,
        ],
    )
    def k(ysw_hbm, inv_hbm, y_hbm, idx_v, buf0, buf1, ob0, ob1,
          sg0, sg1, sw0, sw1):
        wid = lax.axis_index("s") * _NC + lax.axis_index("c")
        tbase = wid * tpw
        bufs = (buf0, buf1)
        obs = (ob0, ob1)
        sgs = (sg0, sg1)
        sws = (sw0, sw1)
        pltpu.sync_copy(inv_hbm.at[pl.ds(2 * tbase, 2 * tpw)], idx_v)
        gathers = [None, None]
        writes = [None, None]
        gathers[0] = pltpu.async_copy(
            ysw_hbm.at[idx_v.at[pl.ds(0, 2 * ct)]], buf0, sg0)
        for c in range(nch):
            s = c % 2
            gathers[s].wait()
            if c + 1 < nch:
                s1 = (c + 1) % 2
                gathers[s1] = pltpu.async_copy(
                    ysw_hbm.at[idx_v.at[pl.ds((c + 1) * 2 * ct, 2 * ct)]],
                    bufs[s1], sgs[s1])
            if writes[s] is not None:
                writes[s].wait()
            buf = bufs[s]
            ob = obs[s]

            def body(i, carry, buf=buf, ob=ob):
                j = i // 4
                vb = (i % 4) * 256
                for u in range(16):
                    v = vb + u * 16
                    ob[j, pl.ds(v, 16)] = (
                        buf[2 * j, pl.ds(v, 16)]
                        + buf[2 * j + 1, pl.ds(v, 16)])
                return carry

            lax.fori_loop(0, nvec // 16, body, 0)
            writes[s] = pltpu.async_copy(
                ob, y_hbm.at[pl.ds(tbase + c * ct, ct)], sws[s])
        writes[0].wait()
        writes[1].wait()

    return k(ysw, inv)


def _gmm_body(blk_r, eid_r, lo_r, hi_r, first_r,
              xs_r, w1_r, w2_r, swb_r, out_r):
    i = pl.program_id(0)
    lo = lo_r[i]
    hi = hi_r[i]
    rows = lax.broadcasted_iota(jnp.int32, (BT, 1), 0)
    mask = (rows >= lo) & (rows < hi)
    xb = jnp.where(mask, xs_r[...], 0.0)
    h = jnp.dot(xb, w1_r[0], preferred_element_type=jnp.float32)
    yv = h[:, :H]
    g = h[:, H:]
    act = yv * (g * jax.nn.sigmoid(g))
    o = jnp.dot(act, w2_r[0], preferred_element_type=jnp.float32)
    o = o * swb_r[:, :1]

    @pl.when(first_r[i] != 0)
    def _():
        out_r[...] = o

    @pl.when(first_r[i] == 0)
    def _():
        out_r[...] = out_r[...] + o


def _grouped_mlp(xs, W1, W2, swb, blk_t, eid_t, lo_t, hi_t, first_t):
    grid_spec = pltpu.PrefetchScalarGridSpec(
        num_scalar_prefetch=5,
        grid=(MAXT,),
        in_specs=[
            pl.BlockSpec((BT, D), lambda i, b, e, l, h, f: (b[i], 0)),
            pl.BlockSpec((1, D, 2 * H), lambda i, b, e, l, h, f: (e[i], 0, 0)),
            pl.BlockSpec((1, H, D), lambda i, b, e, l, h, f: (e[i], 0, 0)),
            pl.BlockSpec((BT, 128), lambda i, b, e, l, h, f: (b[i], 0)),
        ],
        out_specs=pl.BlockSpec((BT, D), lambda i, b, e, l, h, f: (b[i], 0)),
    )
    return pl.pallas_call(
        _gmm_body,
        grid_spec=grid_spec,
        out_shape=jax.ShapeDtypeStruct((N, D), jnp.float32),
    )(blk_t, eid_t, lo_t, hi_t, first_t, xs, W1, W2, swb)


def kernel(x, weights, indices, W1, W2):
    tok, sw, inv, blk_t, eid_t, lo_t, hi_t, first_t = _routing_metadata(
        weights, indices)
    swb = jnp.broadcast_to(sw[:, None], (N, 128))
    xs = _sc_gather(x, tok)
    ysw = _grouped_mlp(xs, W1, W2, swb, blk_t, eid_t, lo_t, hi_t, first_t)
    return _sc_pair_add(ysw, inv)


# speedup vs baseline: 1.4563x; 1.2307x over previous
import functools

import jax
import jax.numpy as jnp
from jax import lax
from jax.experimental import pallas as pl
from jax.experimental.pallas import tpu as pltpu
from jax.experimental.pallas import tpu_sc as plsc

E = 16
T = 2048
D = 1024
H = 1024
K = 2
N = T * K
BT = 256
NB = N // BT
MAXT = NB + E - 1

_NC = 2
_NS = 16
_NW = _NC * _NS


def _routing_metadata(weights, indices):
    eid = indices.reshape(-1).astype(jnp.int32)
    order = jnp.argsort(eid, stable=True).astype(jnp.int32)
    tok = (order // K).astype(jnp.int32)
    sw = weights.reshape(-1)[order]
    inv = jnp.argsort(order).astype(jnp.int32)
    offs = jnp.sum(
        eid[None, :] < jnp.arange(E + 1, dtype=jnp.int32)[:, None],
        axis=1, dtype=jnp.int32)
    counts = offs[1:] - offs[:E]
    fb = offs[:E] // BT
    lb = jnp.maximum(offs[1:] - 1, 0) // BT
    tiles_per = jnp.where(counts > 0, lb - fb + 1, 0).astype(jnp.int32)
    toffs = jnp.concatenate(
        [jnp.zeros((1,), jnp.int32), jnp.cumsum(tiles_per).astype(jnp.int32)])
    total = toffs[E]
    ti = jnp.arange(MAXT, dtype=jnp.int32)
    e_raw = jnp.clip(
        jnp.searchsorted(toffs, ti, side="right",
                         method="compare_all").astype(jnp.int32) - 1,
        0, E - 1)
    b_raw = fb[e_raw] + (ti - toffs[e_raw])
    valid = ti < total
    e_last = jnp.max(jnp.where(counts > 0, jnp.arange(E, dtype=jnp.int32), -1))
    eid_t = jnp.where(valid, e_raw, e_last).astype(jnp.int32)
    blk_t = jnp.where(valid, b_raw, NB - 1).astype(jnp.int32)
    lo_g = jnp.maximum(offs[eid_t], blk_t * BT)
    hi_g = jnp.minimum(offs[eid_t + 1], (blk_t + 1) * BT)
    lo_t = jnp.where(valid, lo_g - blk_t * BT, 0).astype(jnp.int32)
    hi_t = jnp.where(valid, hi_g - blk_t * BT, 0).astype(jnp.int32)
    first_t = jnp.concatenate(
        [jnp.ones((1,), jnp.int32),
         (blk_t[1:] != blk_t[:-1]).astype(jnp.int32)])
    run_start = jnp.concatenate(
        [jnp.ones((1,), jnp.int32),
         (eid_t[1:] != eid_t[:-1]).astype(jnp.int32)])
    run_id = jnp.cumsum(run_start).astype(jnp.int32) - 1
    nruns = run_id[MAXT - 1] + 1
    rex = jnp.zeros((MAXT,), jnp.int32).at[run_id].set(eid_t)
    nr = run_id + 1
    wait_t = run_start
    slot_t = run_id % 2
    pref_t = (run_start * (nr < nruns)).astype(jnp.int32)
    prefe_t = rex[jnp.minimum(nr, MAXT - 1)]
    prefs_t = nr % 2
    return (tok, sw, inv, blk_t, eid_t, lo_t, hi_t, first_t,
            wait_t, slot_t, pref_t, prefe_t, prefs_t)


def _sc_gather(table, idx):
    B = idx.shape[0]
    Dd = table.shape[1]
    bpw = B // _NW
    ch = 32
    nch = bpw // ch
    mesh = plsc.VectorSubcoreMesh(core_axis_name="c", subcore_axis_name="s")

    @functools.partial(
        pl.kernel,
        mesh=mesh,
        out_type=jax.ShapeDtypeStruct((B, Dd), table.dtype),
        scratch_types=[
            pltpu.VMEM((bpw,), jnp.int32),
            pltpu.VMEM((ch, Dd), table.dtype),
            pltpu.VMEM((ch, Dd), table.dtype),
            pltpu.SemaphoreType.DMA,
            pltpu.SemaphoreType.DMA,
            pltpu.SemaphoreType.DMA,
            pltpu.SemaphoreType.DMA,
        ],
    )
    def k(table_hbm, idx_hbm, out_hbm, idx_v, buf0, buf1, sg0, sg1, sw0, sw1):
        wid = lax.axis_index("s") * _NC + lax.axis_index("c")
        base = wid * bpw
        bufs = (buf0, buf1)
        sgs = (sg0, sg1)
        sws = (sw0, sw1)
        pltpu.sync_copy(idx_hbm.at[pl.ds(base, bpw)], idx_v)
        writes = [None, None]
        for c in range(nch):
            s = c % 2
            if writes[s] is not None:
                writes[s].wait()
            pltpu.async_copy(
                table_hbm.at[idx_v.at[pl.ds(c * ch, ch)]], bufs[s], sgs[s]
            ).wait()
            writes[s] = pltpu.async_copy(
                bufs[s], out_hbm.at[pl.ds(base + c * ch, ch)], sws[s])
        writes[0].wait()
        writes[1].wait()

    return k(table, idx)


def _sc_pair_add(ysw, inv):
    tpw = T // _NW
    ct = 16
    nch = tpw // ct
    nvec = ct * (D // 16)
    mesh = plsc.VectorSubcoreMesh(core_axis_name="c", subcore_axis_name="s")

    @functools.partial(
        pl.kernel,
        mesh=mesh,
        out_type=jax.ShapeDtypeStruct((T, D), jnp.float32),
        scratch_types=[
            pltpu.VMEM((2 * tpw,), jnp.int32),
            pltpu.VMEM((2 * ct, D), jnp.float32),
            pltpu.VMEM((2 * ct, D), jnp.float32),
            pltpu.VMEM((ct, D), jnp.float32),
            pltpu.VMEM((ct, D), jnp.float32),
            pltpu.SemaphoreType.DMA,
            pltpu.SemaphoreType.DMA,
            pltpu.SemaphoreType.DMA,
            pltpu.SemaphoreType.DMA,
        ],
    )
    def k(ysw_hbm, inv_hbm, y_hbm, idx_v, buf0, buf1, ob0, ob1,
          sg0, sg1, sw0, sw1):
        wid = lax.axis_index("s") * _NC + lax.axis_index("c")
        tbase = wid * tpw
        bufs = (buf0, buf1)
        obs = (ob0, ob1)
        sgs = (sg0, sg1)
        sws = (sw0, sw1)
        pltpu.sync_copy(inv_hbm.at[pl.ds(2 * tbase, 2 * tpw)], idx_v)
        gathers = [None, None]
        writes = [None, None]
        gathers[0] = pltpu.async_copy(
            ysw_hbm.at[idx_v.at[pl.ds(0, 2 * ct)]], buf0, sg0)
        for c in range(nch):
            s = c % 2
            gathers[s].wait()
            if c + 1 < nch:
                s1 = (c + 1) % 2
                gathers[s1] = pltpu.async_copy(
                    ysw_hbm.at[idx_v.at[pl.ds((c + 1) * 2 * ct, 2 * ct)]],
                    bufs[s1], sgs[s1])
            if writes[s] is not None:
                writes[s].wait()
            buf = bufs[s]
            ob = obs[s]

            def body(i, carry, buf=buf, ob=ob):
                j = i // 4
                vb = (i % 4) * 256
                for u in range(16):
                    v = vb + u * 16
                    ob[j, pl.ds(v, 16)] = (
                        buf[2 * j, pl.ds(v, 16)]
                        + buf[2 * j + 1, pl.ds(v, 16)])
                return carry

            lax.fori_loop(0, nvec // 16, body, 0)
            writes[s] = pltpu.async_copy(
                ob, y_hbm.at[pl.ds(tbase + c * ct, ct)], sws[s])
        writes[0].wait()
        writes[1].wait()

    return k(ysw, inv)


def _gmm_body(blk_r, eid_r, lo_r, hi_r, first_r,
              wait_r, slot_r, pref_r, prefe_r, prefs_r,
              xs_r, w1_hbm, w2_hbm, swb_r, out_r,
              w1b, w2b, sem1, sem2):
    i = pl.program_id(0)

    @pl.when(i == 0)
    def _():
        e0 = eid_r[0]
        pltpu.make_async_copy(w1_hbm.at[e0], w1b.at[0], sem1.at[0]).start()
        pltpu.make_async_copy(w2_hbm.at[e0], w2b.at[0], sem2.at[0]).start()

    @pl.when(pref_r[i] != 0)
    def _():
        e = prefe_r[i]
        s = prefs_r[i]
        pltpu.make_async_copy(w1_hbm.at[e], w1b.at[s], sem1.at[s]).start()
        pltpu.make_async_copy(w2_hbm.at[e], w2b.at[s], sem2.at[s]).start()

    @pl.when(wait_r[i] != 0)
    def _():
        s = slot_r[i]
        pltpu.make_async_copy(w1_hbm.at[0], w1b.at[s], sem1.at[s]).wait()
        pltpu.make_async_copy(w2_hbm.at[0], w2b.at[s], sem2.at[s]).wait()

    cs = slot_r[i]
    lo = lo_r[i]
    hi = hi_r[i]
    rows = lax.broadcasted_iota(jnp.int32, (BT, 1), 0)
    mask = (rows >= lo) & (rows < hi)
    xb = jnp.where(mask, xs_r[...], 0.0)
    h = jnp.dot(xb, w1b[cs], preferred_element_type=jnp.float32)
    yv = h[:, :H]
    g = h[:, H:]
    act = yv * (g * jax.nn.sigmoid(g))
    o = jnp.dot(act, w2b[cs], preferred_element_type=jnp.float32)
    o = o * swb_r[:, :1]

    @pl.when(first_r[i] != 0)
    def _():
        out_r[...] = o

    @pl.when(first_r[i] == 0)
    def _():
        out_r[...] = out_r[...] + o


def _grouped_mlp(xs, W1, W2, swb, blk_t, eid_t, lo_t, hi_t, first_t,
                 wait_t, slot_t, pref_t, prefe_t, prefs_t):
    grid_spec = pltpu.PrefetchScalarGridSpec(
        num_scalar_prefetch=10,
        grid=(MAXT,),
        in_specs=[
            pl.BlockSpec((BT, D), lambda i, *s: (s[0][i], 0)),
            pl.BlockSpec(memory_space=pl.ANY),
            pl.BlockSpec(memory_space=pl.ANY),
            pl.BlockSpec((BT, 128), lambda i, *s: (s[0][i], 0)),
        ],
        out_specs=pl.BlockSpec((BT, D), lambda i, *s: (s[0][i], 0)),
        scratch_shapes=[
            pltpu.VMEM((2, D, 2 * H), jnp.float32),
            pltpu.VMEM((2, H, D), jnp.float32),
            pltpu.SemaphoreType.DMA((2,)),
            pltpu.SemaphoreType.DMA((2,)),
        ],
    )
    return pl.pallas_call(
        _gmm_body,
        grid_spec=grid_spec,
        out_shape=jax.ShapeDtypeStruct((N, D), jnp.float32),
    )(blk_t, eid_t, lo_t, hi_t, first_t,
      wait_t, slot_t, pref_t, prefe_t, prefs_t,
      xs, W1, W2, swb)


def kernel(x, weights, indices, W1, W2):
    (tok, sw, inv, blk_t, eid_t, lo_t, hi_t, first_t,
     wait_t, slot_t, pref_t, prefe_t, prefs_t) = _routing_metadata(
        weights, indices)
    swb = jnp.broadcast_to(sw[:, None], (N, 128))
    xs = _sc_gather(x, tok)
    ysw = _grouped_mlp(xs, W1, W2, swb, blk_t, eid_t, lo_t, hi_t, first_t,
                       wait_t, slot_t, pref_t, prefe_t, prefs_t)
    return _sc_pair_add(ysw, inv)
